# Initial kernel scaffold; baseline (speedup 1.0000x reference)
import jax
import jax.numpy as jnp
from jax.experimental import pallas as pl


def kernel(x, edge_index, t1_w, t1_b, t2_w, t2_b, att_l_w, att_l_b, att_r_w, att_r_b):
    raise NotImplementedError("write your pallas kernel here")


# speedup vs baseline: 22.8405x; 22.8405x over previous
import functools

import jax
import jax.numpy as jnp
from jax import lax
from jax.experimental import pallas as pl
from jax.experimental.pallas import tpu as pltpu
from jax.experimental.pallas import tpu_sc as plsc

N = 10000
NPAD = 10240
E = 320000
D_IN = 128
H = 64
D_OUT = 40
EPS = 0.2

B = 128
EPT = 10112
CH = EPT // B
EPAD = 32 * EPT
ROWS_PER_TILE = NPAD // 16

_mesh = plsc.VectorSubcoreMesh(core_axis_name="c", subcore_axis_name="s")
_sc_params = pltpu.CompilerParams(needs_layout_passes=False,
                                  use_tc_tiling_on_sc=False)


@functools.partial(
    pl.kernel,
    out_type=jax.ShapeDtypeStruct((2, NPAD), jnp.float32),
    mesh=_mesh,
    compiler_params=_sc_params,
    scratch_types=[
        pltpu.VMEM((B,), jnp.int32),
        pltpu.VMEM((B,), jnp.float32),
        pltpu.VMEM((ROWS_PER_TILE,), jnp.float32),
        pltpu.VMEM_SHARED((NPAD,), jnp.float32),
        pltpu.SemaphoreType.DMA,
    ],
)
def _sc_deg(dst_hbm, out_hbm, dst_buf, ones, zbuf, dacc, sem):
    c = lax.axis_index("c")
    s = lax.axis_index("s")
    wid = c * 16 + s
    for g in range(B // 16):
        ones[pl.ds(g * 16, 16)] = jnp.ones((16,), jnp.float32)

    def zb(i, carry):
        zbuf[pl.ds(i * 16, 16)] = jnp.zeros((16,), jnp.float32)
        return carry

    lax.fori_loop(0, ROWS_PER_TILE // 16, zb, 0)
    pltpu.sync_copy(zbuf, dacc.at[pl.ds(s * ROWS_PER_TILE, ROWS_PER_TILE)])
    plsc.subcore_barrier()

    ebase = wid * EPT

    def chunk(k, carry):
        base = ebase + k * B
        pltpu.sync_copy(dst_hbm.at[pl.ds(base, B)], dst_buf)
        pltpu.sync_copy(ones, dacc.at[dst_buf], add=True)
        return carry

    lax.fori_loop(0, CH, chunk, 0)
    plsc.subcore_barrier()
    pltpu.sync_copy(
        dacc.at[pl.ds(s * ROWS_PER_TILE, ROWS_PER_TILE)],
        out_hbm.at[c, pl.ds(s * ROWS_PER_TILE, ROWS_PER_TILE)],
    )


@functools.partial(
    pl.kernel,
    out_type=jax.ShapeDtypeStruct((2, NPAD, H), jnp.float32),
    mesh=_mesh,
    compiler_params=_sc_params,
    scratch_types=[
        pltpu.VMEM((NPAD,), jnp.float32),
        pltpu.VMEM((NPAD,), jnp.float32),
        pltpu.VMEM((NPAD,), jnp.float32),
        pltpu.VMEM((B,), jnp.int32),
        pltpu.VMEM((B,), jnp.int32),
        pltpu.VMEM((B,), jnp.float32),
        pltpu.VMEM((B, H), jnp.float32),
        pltpu.VMEM_SHARED((NPAD, H), jnp.float32),
        pltpu.SemaphoreType.DMA,
    ],
)
def _sc_msg(h_hbm, al_hbm, ar_hbm, dis_hbm, src_hbm, dst_hbm, out_hbm,
            al_buf, ar_buf, dis_buf, src_buf, dst_buf, w_buf, rows, acc, sem):
    c = lax.axis_index("c")
    s = lax.axis_index("s")
    wid = c * 16 + s

    def zr(r, carry):
        for j in range(H // 16):
            rows[r, pl.ds(j * 16, 16)] = jnp.zeros((16,), jnp.float32)
        return carry

    lax.fori_loop(0, B, zr, 0)
    for t in range(ROWS_PER_TILE // B):
        pltpu.sync_copy(rows, acc.at[pl.ds(s * ROWS_PER_TILE + t * B, B)])
    pltpu.sync_copy(al_hbm, al_buf)
    pltpu.sync_copy(ar_hbm, ar_buf)
    pltpu.sync_copy(dis_hbm, dis_buf)
    plsc.subcore_barrier()

    ebase = wid * EPT

    def chunk(k, carry):
        base = ebase + k * B
        pltpu.sync_copy(src_hbm.at[pl.ds(base, B)], src_buf)
        pltpu.sync_copy(dst_hbm.at[pl.ds(base, B)], dst_buf)
        for g in range(B // 16):
            sv = src_buf[pl.ds(g * 16, 16)]
            dv = dst_buf[pl.ds(g * 16, 16)]
            alv = plsc.load_gather(al_buf, [sv])
            arv = plsc.load_gather(ar_buf, [dv])
            dsv = plsc.load_gather(dis_buf, [sv])
            ddv = plsc.load_gather(dis_buf, [dv])
            sarg = alv + arv
            e2 = jnp.exp(sarg + sarg)
            th = 1.0 - 2.0 / (e2 + 1.0)
            w_buf[pl.ds(g * 16, 16)] = th * dsv * ddv
        pltpu.async_copy(h_hbm.at[src_buf], rows, sem).wait()

        def scale(r, carry2):
            wsp = plsc.load_gather(w_buf, [jnp.full((16,), r, jnp.int32)])
            rr = rows.at[r]
            for j in range(H // 16):
                rr[pl.ds(j * 16, 16)] = rr[pl.ds(j * 16, 16)] * wsp
            return carry2

        lax.fori_loop(0, B, scale, 0)
        pltpu.sync_copy(rows, acc.at[dst_buf], add=True)
        return carry

    lax.fori_loop(0, CH, chunk, 0)
    plsc.subcore_barrier()
    pltpu.sync_copy(
        acc.at[pl.ds(s * ROWS_PER_TILE, ROWS_PER_TILE)],
        out_hbm.at[c, pl.ds(s * ROWS_PER_TILE, ROWS_PER_TILE)],
    )


_RB = 1024
_GRID = NPAD // _RB


def _dense_in(x_pad, w1t, b1r, att_a, att_b, degp):
    def body(x_ref, w_ref, b_ref, aa_ref, ab_ref, dg_ref, h_ref, alr_ref, dis_ref):
        hb = jnp.maximum(
            jnp.dot(x_ref[...], w_ref[...], preferred_element_type=jnp.float32)
            + b_ref[...], 0.0)
        h_ref[...] = hb
        alr_ref[...] = (
            jnp.dot(hb, aa_ref[...], preferred_element_type=jnp.float32)
            + ab_ref[...])
        dg = dg_ref[0:1, :] + dg_ref[1:2, :]
        pos = dg > 0.0
        dis_ref[...] = jnp.where(pos, lax.rsqrt(jnp.where(pos, dg, 1.0)), 0.0)

    return pl.pallas_call(
        body,
        grid=(_GRID,),
        in_specs=[
            pl.BlockSpec((_RB, D_IN), lambda i: (i, 0)),
            pl.BlockSpec((D_IN, H), lambda i: (0, 0)),
            pl.BlockSpec((1, H), lambda i: (0, 0)),
            pl.BlockSpec((H, 8), lambda i: (0, 0)),
            pl.BlockSpec((1, 8), lambda i: (0, 0)),
            pl.BlockSpec((2, _RB), lambda i: (0, i)),
        ],
        out_specs=[
            pl.BlockSpec((_RB, H), lambda i: (i, 0)),
            pl.BlockSpec((_RB, 8), lambda i: (i, 0)),
            pl.BlockSpec((1, _RB), lambda i: (0, i)),
        ],
        out_shape=[
            jax.ShapeDtypeStruct((NPAD, H), jnp.float32),
            jax.ShapeDtypeStruct((NPAD, 8), jnp.float32),
            jax.ShapeDtypeStruct((1, NPAD), jnp.float32),
        ],
    )(x_pad, w1t, b1r, att_a, att_b, degp)


def _dense_mid(part, raw, att_a, att_b):
    def body(p_ref, raw_ref, aa_ref, ab_ref, h2_ref, alr_ref):
        h2 = p_ref[0] + p_ref[1] + EPS * raw_ref[...]
        h2_ref[...] = h2
        alr_ref[...] = (
            jnp.dot(h2, aa_ref[...], preferred_element_type=jnp.float32)
            + ab_ref[...])

    return pl.pallas_call(
        body,
        grid=(_GRID,),
        in_specs=[
            pl.BlockSpec((2, _RB, H), lambda i: (0, i, 0)),
            pl.BlockSpec((_RB, H), lambda i: (i, 0)),
            pl.BlockSpec((H, 8), lambda i: (0, 0)),
            pl.BlockSpec((1, 8), lambda i: (0, 0)),
        ],
        out_specs=[
            pl.BlockSpec((_RB, H), lambda i: (i, 0)),
            pl.BlockSpec((_RB, 8), lambda i: (i, 0)),
        ],
        out_shape=[
            jax.ShapeDtypeStruct((NPAD, H), jnp.float32),
            jax.ShapeDtypeStruct((NPAD, 8), jnp.float32),
        ],
    )(part, raw, att_a, att_b)


def _dense_out(part, raw, w2t, b2r):
    def body(p_ref, raw_ref, w_ref, b_ref, out_ref, emb_ref):
        h3 = p_ref[0] + p_ref[1] + EPS * raw_ref[...]
        emb = (jnp.dot(h3, w_ref[...], preferred_element_type=jnp.float32)
               + b_ref[...])
        m = jnp.max(emb, axis=1, keepdims=True)
        ex = jnp.exp(emb - m)
        lse = jnp.log(jnp.sum(ex, axis=1, keepdims=True)) + m
        emb_ref[...] = emb
        out_ref[...] = emb - lse

    return pl.pallas_call(
        body,
        grid=(_GRID,),
        in_specs=[
            pl.BlockSpec((2, _RB, H), lambda i: (0, i, 0)),
            pl.BlockSpec((_RB, H), lambda i: (i, 0)),
            pl.BlockSpec((H, D_OUT), lambda i: (0, 0)),
            pl.BlockSpec((1, D_OUT), lambda i: (0, 0)),
        ],
        out_specs=[
            pl.BlockSpec((_RB, D_OUT), lambda i: (i, 0)),
            pl.BlockSpec((_RB, D_OUT), lambda i: (i, 0)),
        ],
        out_shape=[
            jax.ShapeDtypeStruct((NPAD, D_OUT), jnp.float32),
            jax.ShapeDtypeStruct((NPAD, D_OUT), jnp.float32),
        ],
    )(part, raw, w2t, b2r)


def _att8(att_l_w, att_l_b, att_r_w, att_r_b, layer):
    a = jnp.stack([att_l_w[layer], att_r_w[layer]], axis=1)
    a = jnp.pad(a, ((0, 0), (0, 6)))
    b = jnp.pad(jnp.stack([att_l_b[layer], att_r_b[layer]])[None, :],
                ((0, 0), (0, 6)))
    return a, b


def kernel(x, edge_index, t1_w, t1_b, t2_w, t2_b,
           att_l_w, att_l_b, att_r_w, att_r_b):
    src = edge_index[0]
    dst = edge_index[1]
    pad_ids = (jnp.arange(EPAD - E, dtype=jnp.int32) % (NPAD - N)) + N
    src_pad = jnp.concatenate([src, pad_ids])
    dst_pad = jnp.concatenate([dst, pad_ids])
    x_pad = jnp.concatenate(
        [x, jnp.zeros((NPAD - N, D_IN), jnp.float32)], axis=0)
    w1t = t1_w.T
    b1r = t1_b[None, :]
    w2t = t2_w.T
    b2r = t2_b[None, :]
    aa1, ab1 = _att8(att_l_w, att_l_b, att_r_w, att_r_b, 0)
    aa2, ab2 = _att8(att_l_w, att_l_b, att_r_w, att_r_b, 1)

    degp = _sc_deg(dst_pad)
    h, alr1, dis2 = _dense_in(x_pad, w1t, b1r, aa1, ab1, degp)
    dis = dis2[0]
    part1 = _sc_msg(h, alr1[:, 0], alr1[:, 1], dis, src_pad, dst_pad)
    h2, alr2 = _dense_mid(part1, h, aa2, ab2)
    part2 = _sc_msg(h2, alr2[:, 0], alr2[:, 1], dis, src_pad, dst_pad)
    outp, emb = _dense_out(part2, h, w2t, b2r)
    return outp[:N], emb[:N]
